# Initial kernel scaffold; baseline (speedup 1.0000x reference)
import jax
import jax.numpy as jnp
from jax.experimental import pallas as pl


def kernel(h, idx):
    raise NotImplementedError("write your pallas kernel here")


# speedup vs baseline: 1.3492x; 1.3492x over previous
import functools

import jax
import jax.numpy as jnp
from jax import lax
from jax.experimental import pallas as pl
from jax.experimental.pallas import tpu as pltpu
from jax.experimental.pallas import tpu_sc as plsc

R = 2.0
T = 1.0

N_NODES = 10000
D = 128
N_EDGES = 320000

NC = 2
NS = 16
NW = NC * NS
L = 16

EPW = N_EDGES // NW
K = 400
NCHUNK = EPW // K
NGROUP = K // L


def _renorm_body(h_ref, o_ref):
    h = h_ref[...]
    norm = jnp.sqrt(jnp.sum(h * h, axis=-1, keepdims=True))
    scale = jnp.where(norm > 1.0, 1.0 / (norm + 1e-7), 1.0)
    o_ref[...] = h * scale


def _renorm(h):
    rows = h.shape[0]
    blk = 1000
    return pl.pallas_call(
        _renorm_body,
        grid=(rows // blk,),
        in_specs=[pl.BlockSpec((blk, D), lambda i: (i, 0))],
        out_specs=pl.BlockSpec((blk, D), lambda i: (i, 0)),
        out_shape=jax.ShapeDtypeStruct((rows, D), jnp.float32),
    )(h)


def _decode_body(hn_hbm, idx0_hbm, idx1_hbm, out_hbm,
                 idx0_v, idx1_v, rows0_v, rows1_v, out_v, sem):
    wid = lax.axis_index("s") * NC + lax.axis_index("c")

    def chunk_body(c, carry):
        base = wid * EPW + c * K
        pltpu.sync_copy(idx0_hbm.at[pl.ds(base, K)], idx0_v)
        pltpu.sync_copy(idx1_hbm.at[pl.ds(base, K)], idx1_v)
        cp0 = pltpu.async_copy(hn_hbm.at[idx0_v], rows0_v, sem)
        cp1 = pltpu.async_copy(hn_hbm.at[idx1_v], rows1_v, sem)
        cp0.wait()
        cp1.wait()

        def group_body(g, carry2):
            eidx = g * L + jnp.arange(L, dtype=jnp.int32)
            z = jnp.zeros((L,), jnp.float32)

            def d_body(t, accs):
                d = t * 4
                out = []
                for u in range(4):
                    dv = jnp.full((L,), d + u, dtype=jnp.int32)
                    a = plsc.load_gather(rows0_v, [eidx, dv])
                    b = plsc.load_gather(rows1_v, [eidx, dv])
                    df = a - b
                    out.append(accs[u] + df * df)
                return tuple(out)

            a0, a1, a2, a3 = lax.fori_loop(0, D // 4, d_body, (z, z, z, z))
            sq = (a0 + a1) + (a2 + a3)
            p = 1.0 / (jnp.exp((sq - R) * (1.0 / T)) + 1.0)
            out_v[pl.ds(g * L, L)] = p
            return carry2

        lax.fori_loop(0, NGROUP, group_body, 0)
        pltpu.sync_copy(out_v, out_hbm.at[pl.ds(base, K)])
        return carry

    lax.fori_loop(0, NCHUNK, chunk_body, 0)


@functools.partial(jax.jit, donate_argnums=())
def _decode(hn, idx0, idx1):
    mesh = plsc.VectorSubcoreMesh(core_axis_name="c", subcore_axis_name="s")
    return pl.kernel(
        _decode_body,
        mesh=mesh,
        out_type=jax.ShapeDtypeStruct((N_EDGES,), jnp.float32),
        scratch_types=[
            pltpu.VMEM((K,), jnp.int32),
            pltpu.VMEM((K,), jnp.int32),
            pltpu.VMEM((K, D), jnp.float32),
            pltpu.VMEM((K, D), jnp.float32),
            pltpu.VMEM((K,), jnp.float32),
            pltpu.SemaphoreType.DMA,
        ],
        compiler_params=pltpu.CompilerParams(needs_layout_passes=False),
    )(hn, idx0, idx1)


def kernel(h, idx):
    idx = idx.astype(jnp.int32)
    hn = _renorm(h)
    return _decode(hn, idx[:, 0], idx[:, 1])


# speedup vs baseline: 8.6969x; 6.4459x over previous
import functools

import jax
import jax.numpy as jnp
from jax import lax
from jax.experimental import pallas as pl
from jax.experimental.pallas import tpu as pltpu
from jax.experimental.pallas import tpu_sc as plsc

R = 2.0
T = 1.0

N_NODES = 10000
D = 128
N_EDGES = 320000

NC = 2
NS = 16
NW = NC * NS
L = 16

EPW = N_EDGES // NW
K = 80
NCHUNK = EPW // K


def _renorm_body(h_ref, o_ref):
    h = h_ref[...]
    norm = jnp.sqrt(jnp.sum(h * h, axis=-1, keepdims=True))
    scale = jnp.where(norm > 1.0, 1.0 / (norm + 1e-7), 1.0)
    o_ref[...] = h * scale


def _renorm(h):
    rows = h.shape[0]
    blk = 1000
    return pl.pallas_call(
        _renorm_body,
        grid=(rows // blk,),
        in_specs=[pl.BlockSpec((blk, D), lambda i: (i, 0))],
        out_specs=pl.BlockSpec((blk, D), lambda i: (i, 0)),
        out_shape=jax.ShapeDtypeStruct((rows, D), jnp.float32),
    )(h)


def _decode_body(hn_hbm, idx0_hbm, idx1_hbm, out_hbm,
                 idx0_v, idx1_v, r0a, r1a, r0b, r1b, out_v, semA, semB):
    wid = lax.axis_index("s") * NC + lax.axis_index("c")
    base = wid * EPW

    pltpu.sync_copy(idx0_hbm.at[pl.ds(base, EPW)], idx0_v)
    pltpu.sync_copy(idx1_hbm.at[pl.ds(base, EPW)], idx1_v)

    def start(c, r0, r1, sem):
        pltpu.async_copy(hn_hbm.at[idx0_v.at[pl.ds(c * K, K)]], r0, sem)
        pltpu.async_copy(hn_hbm.at[idx1_v.at[pl.ds(c * K, K)]], r1, sem)

    def wait(r0, r1, sem):
        pltpu.make_async_copy(hn_hbm.at[pl.ds(0, K)], r0, sem).wait()
        pltpu.make_async_copy(hn_hbm.at[pl.ds(0, K)], r1, sem).wait()

    lane15 = jnp.arange(L, dtype=jnp.int32) == (L - 1)

    def compute(c, r0, r1):
        cbase = c * K

        @plsc.parallel_loop(0, K, unroll=2)
        def edge_body(e):
            accs = [jnp.zeros((L,), jnp.float32) for _ in range(4)]
            for d in range(D // L):
                a = r0[e, pl.ds(d * L, L)]
                b = r1[e, pl.ds(d * L, L)]
                df = a - b
                accs[d % 4] = accs[d % 4] + df * df
            sqv = plsc.cumsum((accs[0] + accs[1]) + (accs[2] + accs[3]))
            pos = jnp.full((L,), cbase + e, dtype=jnp.int32)
            plsc.store_scatter(out_v, [pos], sqv, mask=lane15)

    start(0, r0a, r1a, semA)
    start(1, r0b, r1b, semB)

    def pair_body(j, carry):
        c0 = 2 * j
        wait(r0a, r1a, semA)
        compute(c0, r0a, r1a)

        @pl.when(c0 + 2 < NCHUNK)
        def _():
            start(c0 + 2, r0a, r1a, semA)

        wait(r0b, r1b, semB)
        compute(c0 + 1, r0b, r1b)

        @pl.when(c0 + 3 < NCHUNK)
        def _():
            start(c0 + 3, r0b, r1b, semB)

        return carry

    lax.fori_loop(0, NCHUNK // 2, pair_body, 0)
    wait(r0a, r1a, semA)
    compute(NCHUNK - 1, r0a, r1a)

    @plsc.parallel_loop(0, EPW // L, unroll=2)
    def prob_body(g):
        sq = out_v[pl.ds(g * L, L)]
        out_v[pl.ds(g * L, L)] = 1.0 / (jnp.exp((sq - R) * (1.0 / T)) + 1.0)

    pltpu.sync_copy(out_v, out_hbm.at[pl.ds(base, EPW)])


@jax.jit
def _decode(hn, idx0, idx1):
    mesh = plsc.VectorSubcoreMesh(core_axis_name="c", subcore_axis_name="s")
    return pl.kernel(
        _decode_body,
        mesh=mesh,
        out_type=jax.ShapeDtypeStruct((N_EDGES,), jnp.float32),
        scratch_types=[
            pltpu.VMEM((EPW,), jnp.int32),
            pltpu.VMEM((EPW,), jnp.int32),
            pltpu.VMEM((K, D), jnp.float32),
            pltpu.VMEM((K, D), jnp.float32),
            pltpu.VMEM((K, D), jnp.float32),
            pltpu.VMEM((K, D), jnp.float32),
            pltpu.VMEM((EPW,), jnp.float32),
            pltpu.SemaphoreType.DMA,
            pltpu.SemaphoreType.DMA,
        ],
        compiler_params=pltpu.CompilerParams(needs_layout_passes=False),
    )(hn, idx0, idx1)


def kernel(h, idx):
    idx = idx.astype(jnp.int32)
    hn = _renorm(h)
    return _decode(hn, idx[:, 0], idx[:, 1])


# speedup vs baseline: 9.7138x; 1.1169x over previous
import functools

import jax
import jax.numpy as jnp
from jax import lax
from jax.experimental import pallas as pl
from jax.experimental.pallas import tpu as pltpu
from jax.experimental.pallas import tpu_sc as plsc

R = 2.0
T = 1.0

N_NODES = 10000
D = 128
N_EDGES = 320000

NC = 2
NS = 16
NW = NC * NS
L = 16

EPW = N_EDGES // NW
K = 80
NCHUNK = EPW // K


def _renorm_body(h_ref, o_ref):
    h = h_ref[...]
    norm = jnp.sqrt(jnp.sum(h * h, axis=-1, keepdims=True))
    scale = jnp.where(norm > 1.0, 1.0 / (norm + 1e-7), 1.0)
    o_ref[...] = (h * scale).astype(jnp.bfloat16)


def _renorm(h):
    rows = h.shape[0]
    blk = 1000
    return pl.pallas_call(
        _renorm_body,
        grid=(rows // blk,),
        in_specs=[pl.BlockSpec((blk, D), lambda i: (i, 0))],
        out_specs=pl.BlockSpec((blk, D), lambda i: (i, 0)),
        out_shape=jax.ShapeDtypeStruct((rows, D), jnp.bfloat16),
    )(h)


def _decode_body(hn_hbm, idx0_hbm, idx1_hbm, out_hbm,
                 idx0_v, idx1_v, r0a, r1a, r0b, r1b, out_v, semA, semB):
    wid = lax.axis_index("s") * NC + lax.axis_index("c")
    base = wid * EPW

    pltpu.sync_copy(idx0_hbm.at[pl.ds(base, EPW)], idx0_v)
    pltpu.sync_copy(idx1_hbm.at[pl.ds(base, EPW)], idx1_v)

    def start(c, r0, r1, sem):
        pltpu.async_copy(hn_hbm.at[idx0_v.at[pl.ds(c * K, K)]], r0, sem)
        pltpu.async_copy(hn_hbm.at[idx1_v.at[pl.ds(c * K, K)]], r1, sem)

    def wait(r0, r1, sem):
        pltpu.make_async_copy(hn_hbm.at[pl.ds(0, K)], r0, sem).wait()
        pltpu.make_async_copy(hn_hbm.at[pl.ds(0, K)], r1, sem).wait()

    lane15 = jnp.arange(L, dtype=jnp.int32) == (L - 1)

    def compute(c, r0, r1):
        cbase = c * K

        @plsc.parallel_loop(0, K, unroll=2)
        def edge_body(e):
            accs = [jnp.zeros((L,), jnp.float32) for _ in range(4)]
            for d in range(D // (2 * L)):
                a = plsc.bitcast(r0[e, pl.ds(d * L, L)], jnp.bfloat16)
                b = plsc.bitcast(r1[e, pl.ds(d * L, L)], jnp.bfloat16)
                df = a - b
                lo, hi = plsc.unpack(df, format=plsc.PackFormat.INTERLEAVED)
                accs[2 * (d % 2)] = accs[2 * (d % 2)] + lo * lo
                accs[2 * (d % 2) + 1] = accs[2 * (d % 2) + 1] + hi * hi
            sqv = plsc.cumsum((accs[0] + accs[1]) + (accs[2] + accs[3]))
            pos = jnp.full((L,), cbase + e, dtype=jnp.int32)
            plsc.store_scatter(out_v, [pos], sqv, mask=lane15)

    start(0, r0a, r1a, semA)
    start(1, r0b, r1b, semB)

    def pair_body(j, carry):
        c0 = 2 * j
        wait(r0a, r1a, semA)
        compute(c0, r0a, r1a)

        @pl.when(c0 + 2 < NCHUNK)
        def _():
            start(c0 + 2, r0a, r1a, semA)

        wait(r0b, r1b, semB)
        compute(c0 + 1, r0b, r1b)

        @pl.when(c0 + 3 < NCHUNK)
        def _():
            start(c0 + 3, r0b, r1b, semB)

        return carry

    lax.fori_loop(0, NCHUNK // 2, pair_body, 0)
    wait(r0a, r1a, semA)
    compute(NCHUNK - 1, r0a, r1a)

    @plsc.parallel_loop(0, EPW // L, unroll=2)
    def prob_body(g):
        sq = out_v[pl.ds(g * L, L)]
        out_v[pl.ds(g * L, L)] = 1.0 / (jnp.exp((sq - R) * (1.0 / T)) + 1.0)

    pltpu.sync_copy(out_v, out_hbm.at[pl.ds(base, EPW)])


@jax.jit
def _decode(hn, idx0, idx1):
    mesh = plsc.VectorSubcoreMesh(core_axis_name="c", subcore_axis_name="s")
    return pl.kernel(
        _decode_body,
        mesh=mesh,
        out_type=jax.ShapeDtypeStruct((N_EDGES,), jnp.float32),
        scratch_types=[
            pltpu.VMEM((EPW,), jnp.int32),
            pltpu.VMEM((EPW,), jnp.int32),
            pltpu.VMEM((K, D // 2), jnp.int32),
            pltpu.VMEM((K, D // 2), jnp.int32),
            pltpu.VMEM((K, D // 2), jnp.int32),
            pltpu.VMEM((K, D // 2), jnp.int32),
            pltpu.VMEM((EPW,), jnp.float32),
            pltpu.SemaphoreType.DMA,
            pltpu.SemaphoreType.DMA,
        ],
        compiler_params=pltpu.CompilerParams(
            needs_layout_passes=False, use_tc_tiling_on_sc=False),
    )(hn, idx0, idx1)


def kernel(h, idx):
    idx = idx.astype(jnp.int32)
    hb = _renorm(h)
    hn32 = jax.lax.bitcast_convert_type(
        hb.reshape(N_NODES, D // 2, 2), jnp.int32)
    return _decode(hn32, idx[:, 0], idx[:, 1])


# speedup vs baseline: 11.1157x; 1.1443x over previous
import functools

import jax
import jax.numpy as jnp
from jax import lax
from jax.experimental import pallas as pl
from jax.experimental.pallas import tpu as pltpu
from jax.experimental.pallas import tpu_sc as plsc

R = 2.0
T = 1.0

N_NODES = 10000
D = 128
N_EDGES = 320000

NC = 2
NS = 16
NW = NC * NS
L = 16

EPW = N_EDGES // NW
K = 200
NCHUNK = EPW // K


def _renorm_body(h_ref, o_ref):
    h = h_ref[...]
    norm = jnp.sqrt(jnp.sum(h * h, axis=-1, keepdims=True))
    scale = jnp.where(norm > 1.0, 1.0 / (norm + 1e-7), 1.0)
    o_ref[...] = (h * scale).astype(jnp.bfloat16)


def _renorm(h):
    rows = h.shape[0]
    blk = 1000
    return pl.pallas_call(
        _renorm_body,
        grid=(rows // blk,),
        in_specs=[pl.BlockSpec((blk, D), lambda i: (i, 0))],
        out_specs=pl.BlockSpec((blk, D), lambda i: (i, 0)),
        out_shape=jax.ShapeDtypeStruct((rows, D), jnp.bfloat16),
    )(h)


def _decode_body(hn_hbm, idx0_hbm, idx1_hbm, out_hbm,
                 idx0_v, idx1_v, r0a, r1a, r0b, r1b, out_v, semA, semB):
    wid = lax.axis_index("s") * NC + lax.axis_index("c")
    base = wid * EPW

    pltpu.sync_copy(idx0_hbm.at[pl.ds(base, EPW)], idx0_v)
    pltpu.sync_copy(idx1_hbm.at[pl.ds(base, EPW)], idx1_v)

    def start(c, r0, r1, sem):
        pltpu.async_copy(hn_hbm.at[idx0_v.at[pl.ds(c * K, K)]], r0, sem)
        pltpu.async_copy(hn_hbm.at[idx1_v.at[pl.ds(c * K, K)]], r1, sem)

    def wait(r0, r1, sem):
        pltpu.make_async_copy(hn_hbm.at[pl.ds(0, K)], r0, sem).wait()
        pltpu.make_async_copy(hn_hbm.at[pl.ds(0, K)], r1, sem).wait()

    lane15 = jnp.arange(L, dtype=jnp.int32) == (L - 1)

    def compute(c, r0, r1):
        cbase = c * K

        @plsc.parallel_loop(0, K, unroll=4)
        def edge_body(e):
            accs = [jnp.zeros((L,), jnp.float32) for _ in range(4)]
            for d in range(D // (2 * L)):
                a = plsc.bitcast(r0[e, pl.ds(d * L, L)], jnp.bfloat16)
                b = plsc.bitcast(r1[e, pl.ds(d * L, L)], jnp.bfloat16)
                df = a - b
                lo, hi = plsc.unpack(df, format=plsc.PackFormat.INTERLEAVED)
                accs[2 * (d % 2)] = accs[2 * (d % 2)] + lo * lo
                accs[2 * (d % 2) + 1] = accs[2 * (d % 2) + 1] + hi * hi
            sqv = plsc.cumsum((accs[0] + accs[1]) + (accs[2] + accs[3]))
            pos = jnp.full((L,), cbase + e, dtype=jnp.int32)
            plsc.store_scatter(out_v, [pos], sqv, mask=lane15)

    start(0, r0a, r1a, semA)
    start(1, r0b, r1b, semB)

    def pair_body(j, carry):
        c0 = 2 * j
        wait(r0a, r1a, semA)
        compute(c0, r0a, r1a)

        @pl.when(c0 + 2 < NCHUNK)
        def _():
            start(c0 + 2, r0a, r1a, semA)

        wait(r0b, r1b, semB)
        compute(c0 + 1, r0b, r1b)

        @pl.when(c0 + 3 < NCHUNK)
        def _():
            start(c0 + 3, r0b, r1b, semB)

        return carry

    lax.fori_loop(0, NCHUNK // 2, pair_body, 0)
    if NCHUNK % 2:
        wait(r0a, r1a, semA)
        compute(NCHUNK - 1, r0a, r1a)

    @plsc.parallel_loop(0, EPW // L, unroll=2)
    def prob_body(g):
        sq = out_v[pl.ds(g * L, L)]
        out_v[pl.ds(g * L, L)] = 1.0 / (jnp.exp((sq - R) * (1.0 / T)) + 1.0)

    pltpu.sync_copy(out_v, out_hbm.at[pl.ds(base, EPW)])


@jax.jit
def _decode(hn, idx0, idx1):
    mesh = plsc.VectorSubcoreMesh(core_axis_name="c", subcore_axis_name="s")
    return pl.kernel(
        _decode_body,
        mesh=mesh,
        out_type=jax.ShapeDtypeStruct((N_EDGES,), jnp.float32),
        scratch_types=[
            pltpu.VMEM((EPW,), jnp.int32),
            pltpu.VMEM((EPW,), jnp.int32),
            pltpu.VMEM((K, D // 2), jnp.int32),
            pltpu.VMEM((K, D // 2), jnp.int32),
            pltpu.VMEM((K, D // 2), jnp.int32),
            pltpu.VMEM((K, D // 2), jnp.int32),
            pltpu.VMEM((EPW,), jnp.float32),
            pltpu.SemaphoreType.DMA,
            pltpu.SemaphoreType.DMA,
        ],
        compiler_params=pltpu.CompilerParams(
            needs_layout_passes=False, use_tc_tiling_on_sc=False),
    )(hn, idx0, idx1)


def kernel(h, idx):
    idx = idx.astype(jnp.int32)
    hb = _renorm(h)
    hn32 = jax.lax.bitcast_convert_type(
        hb.reshape(N_NODES, D // 2, 2), jnp.int32)
    return _decode(hn32, idx[:, 0], idx[:, 1])
